# baseline (speedup 1.0000x reference)
import functools

import jax
import jax.numpy as jnp
import numpy as np
from jax import lax
from jax.experimental import pallas as pl
from jax.experimental.pallas import tpu as pltpu
from jax.experimental.pallas import tpu_sc as plsc

_B = 128
_N = 3 * 224 * 224
_NY = 1000
_NW = 32
_CH = _N // _NW
_V = 16
_VPC = _CH // _V

_key = jax.random.key(42)
_kb, _kp = jax.random.split(_key)
_COEFFS = np.asarray(jax.random.beta(_kb, 0.2, 0.2, (_B,)), np.float32)
_PERM = np.asarray(jax.random.permutation(_kp, _B), np.int32)


def _build_steps():
    seen = np.zeros(_B, bool)
    steps = []
    for s in range(_B):
        if seen[s]:
            continue
        cyc = []
        j = s
        while not seen[j]:
            seen[j] = True
            cyc.append(j)
            j = int(_PERM[j])
        steps.append((cyc[0], None, 0.0))
        for t, o in enumerate(cyc):
            nxt = cyc[(t + 1) % len(cyc)]
            steps.append((nxt, o, float(_COEFFS[o])))
    return steps


_STEPS = _build_steps()
_NS = -(-len(_STEPS) // 6) * 6
_NSP = -(-(_NS + 2) // 8) * 8

_GTAB = np.zeros((_NSP, _V), np.int32)
_OTAB = np.full((_NSP, _V), -1, np.int32)
_CTAB = np.zeros((_NSP, _V), np.float32)
for _t, (_g, _o, _c) in enumerate(_STEPS):
    _GTAB[_t, :] = _g * _NW
    _OTAB[_t, :] = -1 if _o is None else _o * _NW
    _CTAB[_t, :] = _c
_GTAB = _GTAB.reshape(-1)
_OTAB = _OTAB.reshape(-1)
_CTAB = _CTAB.reshape(-1)

_PMAT = np.zeros((_B, _B), np.float32)
_PMAT[np.arange(_B), _PERM] = 1.0


def _sc_x(X4, gtab, otab, ctab):
    mesh = plsc.VectorSubcoreMesh(core_axis_name="c", subcore_axis_name="s")

    @functools.partial(
        pl.kernel,
        mesh=mesh,
        compiler_params=pltpu.CompilerParams(use_tc_tiling_on_sc=False),
        out_type=[
            jax.ShapeDtypeStruct((_B * _NW, _CH), jnp.float32),
            jax.ShapeDtypeStruct((_NW, _CH), jnp.float32),
        ],
        scratch_types=[
            pltpu.VMEM((3 * _CH,), jnp.float32),
            pltpu.VMEM((2 * _CH,), jnp.float32),
            pltpu.VMEM((_NSP * _V,), jnp.int32),
            pltpu.VMEM((_NSP * _V,), jnp.int32),
            pltpu.VMEM((_NSP * _V,), jnp.float32),
            pltpu.SemaphoreType.DMA,
            pltpu.SemaphoreType.DMA,
            pltpu.SemaphoreType.DMA,
            pltpu.SemaphoreType.DMA,
            pltpu.SemaphoreType.DMA,
        ],
    )
    def k(x_hbm, gt_hbm, ot_hbm, ct_hbm, xo_hbm, dump_hbm,
          av, ov, gt_v, ot_v, ct_v, sa0, sa1, sa2, so0, so1):
        wid = lax.axis_index("s") * 2 + lax.axis_index("c")
        sa = (sa0, sa1, sa2)
        so = (so0, so1)
        pltpu.sync_copy(gt_hbm, gt_v)
        pltpu.sync_copy(ot_hbm, ot_v)
        pltpu.sync_copy(ct_hbm, ct_v)
        pltpu.async_copy(ov.at[pl.ds(0, _CH)], dump_hbm.at[wid], so0)
        pltpu.async_copy(ov.at[pl.ds(_CH, _CH)], dump_hbm.at[wid], so1)

        def gather(t, slot):
            gi = gt_v[pl.ds(t * _V, _V)][0]
            pltpu.async_copy(
                x_hbm.at[gi + wid],
                av.at[pl.ds(slot * _CH, _CH)],
                sa[slot],
            )

        def wait_gather(slot):
            pltpu.make_async_copy(
                x_hbm.at[wid], av.at[pl.ds(slot * _CH, _CH)], sa[slot]
            ).wait()

        gather(0, 0)
        gather(1, 1)

        def substep(t, r):
            cur = r % 3
            prv = (r + 2) % 3
            o_slot = r % 2
            wait_gather(cur)
            oi = ot_v[pl.ds(t * _V, _V)][0]

            @pl.when(oi >= 0)
            def _():
                pltpu.make_async_copy(
                    ov.at[pl.ds(o_slot * _CH, _CH)], xo_hbm.at[0],
                    so[o_slot],
                ).wait()
                cvec = ct_v[pl.ds(t * _V, _V)]
                dvec = 1.0 - cvec

                def body(kk, c2):
                    sl = kk * _V
                    ov[pl.ds(o_slot * _CH + sl, _V)] = (
                        cvec * av[pl.ds(prv * _CH + sl, _V)]
                        + dvec * av[pl.ds(cur * _CH + sl, _V)]
                    )
                    return c2

                lax.fori_loop(0, _VPC, body, 0, unroll=14)
                pltpu.async_copy(
                    ov.at[pl.ds(o_slot * _CH, _CH)],
                    xo_hbm.at[oi + wid],
                    so[o_slot],
                )

            gather(t + 2, (r + 2) % 3)

        def six(q, carry):
            base = q * 6
            for r in range(6):
                substep(base + r, r)
            return carry

        lax.fori_loop(0, _NS // 6, six, 0)

        wait_gather(_NS % 3)
        wait_gather((_NS + 1) % 3)
        pltpu.make_async_copy(
            ov.at[pl.ds(0, _CH)], xo_hbm.at[0], so0).wait()
        pltpu.make_async_copy(
            ov.at[pl.ds(_CH, _CH)], xo_hbm.at[0], so1).wait()

    return k(X4, gtab, otab, ctab)[0]


def _y_body(y_ref, p_ref, c_ref, yo_ref):
    y = y_ref[...]
    yp = jnp.dot(p_ref[...], y, preferred_element_type=jnp.float32)
    c = c_ref[...]
    yo_ref[...] = c * y + (1.0 - c) * yp


def kernel(X, Y):
    X4 = X.reshape(_B * _NW, _CH)
    Xo4 = _sc_x(X4, jnp.asarray(_GTAB), jnp.asarray(_OTAB),
                jnp.asarray(_CTAB))
    Yo = pl.pallas_call(
        _y_body,
        out_shape=jax.ShapeDtypeStruct((_B, _NY), jnp.float32),
    )(Y, jnp.asarray(_PMAT), jnp.asarray(_COEFFS.reshape(_B, 1)))
    return Xo4.reshape(X.shape), Yo


# speedup vs baseline: 1.1939x; 1.1939x over previous
import functools

import jax
import jax.numpy as jnp
import numpy as np
from jax import lax
from jax.experimental import pallas as pl
from jax.experimental.pallas import tpu as pltpu
from jax.experimental.pallas import tpu_sc as plsc

_B = 128
_N = 3 * 224 * 224
_NY = 1000
_NW = 32
_CH = _N // _NW
_V = 16
_VPC = _CH // _V

_key = jax.random.key(42)
_kb, _kp = jax.random.split(_key)
_COEFFS = np.asarray(jax.random.beta(_kb, 0.2, 0.2, (_B,)), np.float32)
_PERM = np.asarray(jax.random.permutation(_kp, _B), np.int32)


def _build_steps():
    seen = np.zeros(_B, bool)
    steps = []
    for s in range(_B):
        if seen[s]:
            continue
        cyc = []
        j = s
        while not seen[j]:
            seen[j] = True
            cyc.append(j)
            j = int(_PERM[j])
        steps.append((cyc[0], None, 0.0))
        for t, o in enumerate(cyc):
            nxt = cyc[(t + 1) % len(cyc)]
            steps.append((nxt, o, float(_COEFFS[o])))
    return steps


_STEPS = _build_steps()
_NS = -(-len(_STEPS) // 6) * 6
_NSP = -(-(_NS + 2) // 8) * 8

_GTAB = np.zeros((_NSP, _V), np.int32)
_OTAB = np.full((_NSP, _V), -1, np.int32)
_CTAB = np.zeros((_NSP, _V), np.float32)
for _t, (_g, _o, _c) in enumerate(_STEPS):
    _GTAB[_t, :] = _g * _N
    _OTAB[_t, :] = -1 if _o is None else _o * _N
    _CTAB[_t, :] = _c
_GTAB = _GTAB.reshape(-1)
_OTAB = _OTAB.reshape(-1)
_CTAB = _CTAB.reshape(-1)

_PMAT = np.zeros((_B, _B), np.float32)
_PMAT[np.arange(_B), _PERM] = 1.0


def _sc_x(X4, gtab, otab, ctab):
    mesh = plsc.VectorSubcoreMesh(core_axis_name="c", subcore_axis_name="s")

    @functools.partial(
        pl.kernel,
        mesh=mesh,
        compiler_params=pltpu.CompilerParams(use_tc_tiling_on_sc=False),
        out_type=[
            jax.ShapeDtypeStruct((_B * _N,), jnp.float32),
            jax.ShapeDtypeStruct((_NW * _CH,), jnp.float32),
        ],
        scratch_types=[
            pltpu.VMEM((6 * _CH,), jnp.float32),
            pltpu.VMEM((2 * _CH,), jnp.float32),
            pltpu.VMEM((_NSP * _V,), jnp.int32),
            pltpu.VMEM((_NSP * _V,), jnp.int32),
            pltpu.VMEM((_NSP * _V,), jnp.float32),
            pltpu.SemaphoreType.DMA,
            pltpu.SemaphoreType.DMA,
            pltpu.SemaphoreType.DMA,
            pltpu.SemaphoreType.DMA,
            pltpu.SemaphoreType.DMA,
            pltpu.SemaphoreType.DMA,
            pltpu.SemaphoreType.DMA,
            pltpu.SemaphoreType.DMA,
        ],
    )
    def k(x_hbm, gt_hbm, ot_hbm, ct_hbm, xo_hbm, dump_hbm,
          av, ov, gt_v, ot_v, ct_v, sa0, sa1, sa2, sa3, sa4, sa5, so0, so1):
        wid = lax.axis_index("s") * 2 + lax.axis_index("c")
        woff = pl.multiple_of(wid * _CH, 8)
        sa = (sa0, sa1, sa2, sa3, sa4, sa5)
        so = (so0, so1)
        pltpu.sync_copy(gt_hbm, gt_v)
        pltpu.sync_copy(ot_hbm, ot_v)
        pltpu.sync_copy(ct_hbm, ct_v)
        pltpu.async_copy(ov.at[pl.ds(0, _CH)],
                         dump_hbm.at[pl.ds(pl.multiple_of(woff, 8), _CH)], so0)
        pltpu.async_copy(ov.at[pl.ds(_CH, _CH)],
                         dump_hbm.at[pl.ds(pl.multiple_of(woff, 8), _CH)], so1)

        def gather(t, slot):
            gi = gt_v[pl.ds(pl.multiple_of(t * _V, 8), _V)][0]
            pltpu.async_copy(
                x_hbm.at[pl.ds(pl.multiple_of(gi + woff, 8), _CH)],
                av.at[pl.ds(slot * _CH, _CH)],
                sa[slot],
            )

        def wait_gather(slot):
            pltpu.make_async_copy(
                x_hbm.at[pl.ds(pl.multiple_of(woff, 8), _CH)],
                av.at[pl.ds(slot * _CH, _CH)], sa[slot]
            ).wait()

        gather(0, 0)
        gather(1, 1)
        gather(2, 2)
        gather(3, 3)

        def substep(t, r):
            cur = r
            prv = (r + 5) % 6
            o_slot = r % 2
            wait_gather(cur)
            oi = ot_v[pl.ds(pl.multiple_of(t * _V, 8), _V)][0]

            @pl.when(oi >= 0)
            def _():
                pltpu.make_async_copy(
                    ov.at[pl.ds(o_slot * _CH, _CH)],
                    xo_hbm.at[pl.ds(0, _CH)],
                    so[o_slot],
                ).wait()
                cvec = ct_v[pl.ds(pl.multiple_of(t * _V, 8), _V)]
                dvec = 1.0 - cvec

                def body(kk, c2):
                    sl = pl.multiple_of(kk * _V, 8)
                    ov[pl.ds(o_slot * _CH + sl, _V)] = (
                        cvec * av[pl.ds(prv * _CH + sl, _V)]
                        + dvec * av[pl.ds(cur * _CH + sl, _V)]
                    )
                    return c2

                lax.fori_loop(0, _VPC, body, 0, unroll=14)
                pltpu.async_copy(
                    ov.at[pl.ds(o_slot * _CH, _CH)],
                    xo_hbm.at[pl.ds(pl.multiple_of(oi + woff, 8), _CH)],
                    so[o_slot],
                )

            gather(t + 4, (r + 4) % 6)

        def six(q, carry):
            base = q * 6
            for r in range(6):
                substep(base + r, r)
            return carry

        lax.fori_loop(0, _NS // 6, six, 0)

        wait_gather(_NS % 6)
        wait_gather((_NS + 1) % 6)
        wait_gather((_NS + 2) % 6)
        wait_gather((_NS + 3) % 6)
        pltpu.make_async_copy(
            ov.at[pl.ds(0, _CH)], xo_hbm.at[pl.ds(0, _CH)], so0).wait()
        pltpu.make_async_copy(
            ov.at[pl.ds(_CH, _CH)], xo_hbm.at[pl.ds(_CH, _CH)], so1).wait()

    return k(X4, gtab, otab, ctab)[0]


def _y_body(y_ref, p_ref, c_ref, yo_ref):
    y = y_ref[...]
    yp = jnp.dot(p_ref[...], y, preferred_element_type=jnp.float32)
    c = c_ref[...]
    yo_ref[...] = c * y + (1.0 - c) * yp


def kernel(X, Y):
    Xf = X.reshape(_B * _N)
    Xo4 = _sc_x(Xf, jnp.asarray(_GTAB), jnp.asarray(_OTAB),
                jnp.asarray(_CTAB))
    Yo = pl.pallas_call(
        _y_body,
        out_shape=jax.ShapeDtypeStruct((_B, _NY), jnp.float32),
    )(Y, jnp.asarray(_PMAT), jnp.asarray(_COEFFS.reshape(_B, 1)))
    return Xo4.reshape(X.shape), Yo
